# Initial kernel scaffold; baseline (speedup 1.0000x reference)
import functools

import jax
import jax.numpy as jnp
from jax import lax
from jax.experimental import pallas as pl
from jax.experimental.pallas import tpu as pltpu
from jax.experimental.pallas import tpu_sc as plsc

N = 10000
E = 320000
D = 128
G = 64
NC = 2
NS = 16
L = 16
NW = NC * NS
CHUNK = 128
NCHUNKS = E // CHUNK
NT = -(-NCHUNKS // NW)
SPAN = 624
SCALE = 1.0 / (D ** 0.5)
FP32 = jnp.float32


def _proj_body(x_ref, wq, wk, wv, ws, bq, bk, bv, bs, q_o, k_o, v_o, s_o):
    xb = x_ref[...]
    q_o[...] = jnp.dot(xb, wq[...], preferred_element_type=FP32) + bq[...]
    k_o[...] = jnp.dot(xb, wk[...], preferred_element_type=FP32) + bk[...]
    v_o[...] = jnp.dot(xb, wv[...], preferred_element_type=FP32) + bv[...]
    s_o[...] = jnp.dot(xb, ws[...], preferred_element_type=FP32) + bs[...]


def _project(x, Wq, Wk, Wv, Ws, bq, bk, bv, bs):
    BR = 1000
    grid = (N // BR,)
    row_spec = pl.BlockSpec((BR, D), lambda i: (i, 0))
    w_spec = pl.BlockSpec((D, D), lambda i: (0, 0))
    b_spec = pl.BlockSpec((1, D), lambda i: (0, 0))
    out = jax.ShapeDtypeStruct((N, D), FP32)
    return pl.pallas_call(
        _proj_body,
        grid=grid,
        in_specs=[row_spec, w_spec, w_spec, w_spec, w_spec,
                  b_spec, b_spec, b_spec, b_spec],
        out_specs=[row_spec, row_spec, row_spec, row_spec],
        out_shape=[out, out, out, out],
    )(x, Wq, Wk, Wv, Ws, bq.reshape(1, D), bk.reshape(1, D),
      bv.reshape(1, D), bs.reshape(1, D))


def _edge_body(q_hbm, k_hbm, v_hbm, src_hbm, dst_hbm,
               agg_out, den_out,
               src_idx, dst_idx, qrows, krows, vrows,
               exbuf, partials, zrows, zflat,
               agg_sh, den_sh, sem0, sem1, sem2):
    c = lax.axis_index("c")
    s = lax.axis_index("s")
    wid = s * NC + c

    zero16 = jnp.zeros((L,), FP32)

    def _zrow(r, _):
        for i in range(D // L):
            zrows[r, pl.ds(i * L, L)] = zero16
        return 0

    lax.fori_loop(0, CHUNK, _zrow, 0)
    for i in range(640 // L):
        zflat[pl.ds(i * L, L)] = zero16

    start = s * SPAN
    for j in range(5):
        pltpu.sync_copy(zrows, agg_sh.at[pl.ds(start + j * CHUNK, CHUNK)])
    pltpu.sync_copy(zflat, den_sh.at[pl.ds(start, 640)])
    plsc.subcore_barrier()

    lane_iota = lax.iota(jnp.int32, L)

    def chunk_body(t, _):
        cid = wid + t * NW

        @pl.when(cid < NCHUNKS)
        def _():
            base = cid * CHUNK
            pltpu.sync_copy(src_hbm.at[pl.ds(base, CHUNK)], src_idx)
            pltpu.sync_copy(dst_hbm.at[pl.ds(base, CHUNK)], dst_idx)
            cq = pltpu.async_copy(q_hbm.at[dst_idx], qrows, sem0)
            ck = pltpu.async_copy(k_hbm.at[src_idx], krows, sem1)
            cv = pltpu.async_copy(v_hbm.at[src_idx], vrows, sem2)
            cq.wait()
            ck.wait()
            cv.wait()

            def group_body(g, _):
                gb = g * L
                for j in range(L):
                    e = gb + j
                    acc = qrows[e, pl.ds(0, L)] * krows[e, pl.ds(0, L)]
                    for i in range(1, D // L):
                        acc = acc + (qrows[e, pl.ds(i * L, L)]
                                     * krows[e, pl.ds(i * L, L)])
                    plsc.store_scatter(
                        partials,
                        [lane_iota, jnp.full((L,), j, jnp.int32)], acc)
                score = partials[0, :]
                for l in range(1, L):
                    score = score + partials[l, :]
                ex = jnp.exp(jnp.clip(score * SCALE, -60.0, 60.0))
                exbuf[pl.ds(gb, L)] = ex
                for j in range(L):
                    e = gb + j
                    w = plsc.load_gather(
                        exbuf, [jnp.full((L,), e, jnp.int32)])
                    for i in range(D // L):
                        vrows[e, pl.ds(i * L, L)] = (
                            vrows[e, pl.ds(i * L, L)] * w)
                return 0

            lax.fori_loop(0, CHUNK // L, group_body, 0)
            pltpu.sync_copy(exbuf, den_sh.at[dst_idx], add=True)
            pltpu.sync_copy(vrows, agg_sh.at[dst_idx], add=True)

        return 0

    lax.fori_loop(0, NT, chunk_body, 0)
    plsc.subcore_barrier()

    pltpu.sync_copy(agg_sh.at[pl.ds(start, 640)],
                    agg_out.at[c].at[pl.ds(start, 640)])
    pltpu.sync_copy(den_sh.at[pl.ds(start, 640)],
                    den_out.at[c].at[pl.ds(start, 640)])


def _edge_pass(q, k, v, src, dst):
    mesh = plsc.VectorSubcoreMesh(core_axis_name="c", subcore_axis_name="s")
    call = pl.kernel(
        _edge_body,
        out_type=(jax.ShapeDtypeStruct((NC, N, D), FP32),
                  jax.ShapeDtypeStruct((NC, N), FP32)),
        mesh=mesh,
        scratch_types=[
            pltpu.VMEM((CHUNK,), jnp.int32),
            pltpu.VMEM((CHUNK,), jnp.int32),
            pltpu.VMEM((CHUNK, D), FP32),
            pltpu.VMEM((CHUNK, D), FP32),
            pltpu.VMEM((CHUNK, D), FP32),
            pltpu.VMEM((CHUNK,), FP32),
            pltpu.VMEM((L, L), FP32),
            pltpu.VMEM((CHUNK, D), FP32),
            pltpu.VMEM((640,), FP32),
            pltpu.VMEM_SHARED((N, D), FP32),
            pltpu.VMEM_SHARED((N,), FP32),
            pltpu.SemaphoreType.DMA,
            pltpu.SemaphoreType.DMA,
            pltpu.SemaphoreType.DMA,
        ],
    )
    return call(q, k, v, src, dst)


def _finish_body(aggp_ref, denp_ref, skip_ref, batch_ref, out_ref):
    agg = aggp_ref[0] + aggp_ref[1]
    den = denp_ref[0] + denp_ref[1] + 1e-16
    node = jax.nn.relu(agg / den + skip_ref[...])
    onehot = (batch_ref[...] ==
              lax.broadcasted_iota(jnp.int32, (G, N), 0)).astype(FP32)
    counts = jnp.sum(onehot, axis=1, keepdims=True)
    pooled = jnp.dot(onehot, node, preferred_element_type=FP32)
    out_ref[...] = pooled / jnp.maximum(counts, 1.0)


def _finish(agg_p, den_p, skip, batch):
    return pl.pallas_call(
        _finish_body,
        out_shape=jax.ShapeDtypeStruct((G, D), FP32),
    )(agg_p, den_p.reshape(NC, N, 1), skip, batch.reshape(1, N))


def kernel(x, edge_index, batch, Wq, bq, Wk, bk, Wv, bv, Ws, bs):
    src = edge_index[0]
    dst = edge_index[1]
    q, k, v, skip = _project(x, Wq, Wk, Wv, Ws, bq, bk, bv, bs)
    agg_p, den_p = _edge_pass(q, k, v, src, dst)
    return _finish(agg_p, den_p, skip, batch)


# speedup vs baseline: 11.6530x; 11.6530x over previous
import functools

import jax
import jax.numpy as jnp
from jax import lax
from jax.experimental import pallas as pl
from jax.experimental.pallas import tpu as pltpu
from jax.experimental.pallas import tpu_sc as plsc

N = 10000
E = 320000
D = 128
G = 64
NC = 2
NS = 16
L = 16
NW = NC * NS
CHUNK = 80
NCHUNKS = E // CHUNK
NT = NCHUNKS // NW
SPAN = 624
DEN_PAD = 640 * NS
SCALE = 1.0 / (D ** 0.5)
FP32 = jnp.float32


def _proj_body(x_ref, wq, wk, wv, ws, bq, bk, bv, bs, q_o, k_o, v_o, s_o):
    xb = x_ref[...]
    q_o[...] = jnp.dot(xb, wq[...], preferred_element_type=FP32) + bq[...]
    k_o[...] = jnp.dot(xb, wk[...], preferred_element_type=FP32) + bk[...]
    v_o[...] = jnp.dot(xb, wv[...], preferred_element_type=FP32) + bv[...]
    s_o[...] = jnp.dot(xb, ws[...], preferred_element_type=FP32) + bs[...]


def _project(x, Wq, Wk, Wv, Ws, bq, bk, bv, bs):
    BR = 1000
    grid = (N // BR,)
    row_spec = pl.BlockSpec((BR, D), lambda i: (i, 0))
    w_spec = pl.BlockSpec((D, D), lambda i: (0, 0))
    b_spec = pl.BlockSpec((1, D), lambda i: (0, 0))
    out = jax.ShapeDtypeStruct((N, D), FP32)
    return pl.pallas_call(
        _proj_body,
        grid=grid,
        in_specs=[row_spec, w_spec, w_spec, w_spec, w_spec,
                  b_spec, b_spec, b_spec, b_spec],
        out_specs=[row_spec, row_spec, row_spec, row_spec],
        out_shape=[out, out, out, out],
    )(x, Wq, Wk, Wv, Ws, bq.reshape(1, D), bk.reshape(1, D),
      bv.reshape(1, D), bs.reshape(1, D))


def _edge_body(q_hbm, k_hbm, v_hbm, src_hbm, dst_hbm,
               agg_out, den_out,
               src_idx, dst_idx, qrows, krows, vrows,
               exbuf, partials, zflat,
               agg_sh, den_sh, sem0, sem1, sem2):
    c = lax.axis_index("c")
    s = lax.axis_index("s")
    wid = s * NC + c

    zero16 = jnp.zeros((L,), FP32)

    def _zrow(r, _):
        for i in range(D // L):
            qrows[r, pl.ds(i * L, L)] = zero16
        return 0

    lax.fori_loop(0, CHUNK, _zrow, 0)
    for i in range(640 // L):
        zflat[pl.ds(i * L, L)] = zero16

    start = s * SPAN
    for j in range(640 // CHUNK):
        pltpu.sync_copy(qrows, agg_sh.at[pl.ds(start + j * CHUNK, CHUNK)])
    pltpu.sync_copy(zflat, den_sh.at[pl.ds(s * 640, 640)])
    plsc.subcore_barrier()

    lane_iota = lax.iota(jnp.int32, L)

    def chunk_body(t, _):
        cid = wid + t * NW
        if True:
            base = cid * CHUNK
            pltpu.sync_copy(src_hbm.at[pl.ds(base, CHUNK)], src_idx)
            pltpu.sync_copy(dst_hbm.at[pl.ds(base, CHUNK)], dst_idx)
            cq = pltpu.async_copy(q_hbm.at[dst_idx], qrows, sem0)
            ck = pltpu.async_copy(k_hbm.at[src_idx], krows, sem1)
            cv = pltpu.async_copy(v_hbm.at[src_idx], vrows, sem2)
            cq.wait()
            ck.wait()
            cv.wait()

            def group_body(g, _):
                gb = g * L
                for j in range(L):
                    e = gb + j
                    acc = qrows[e, pl.ds(0, L)] * krows[e, pl.ds(0, L)]
                    for i in range(1, D // L):
                        acc = acc + (qrows[e, pl.ds(i * L, L)]
                                     * krows[e, pl.ds(i * L, L)])
                    plsc.store_scatter(
                        partials, [lane_iota * L + j], acc)
                score = partials[pl.ds(0, L)]
                for l in range(1, L):
                    score = score + partials[pl.ds(l * L, L)]
                ex = jnp.exp(jnp.clip(score * SCALE, -60.0, 60.0))
                exbuf[pl.ds(gb, L)] = ex
                for j in range(L):
                    e = gb + j
                    w = plsc.load_gather(
                        exbuf, [jnp.full((L,), e, jnp.int32)])
                    for i in range(D // L):
                        vrows[e, pl.ds(i * L, L)] = (
                            vrows[e, pl.ds(i * L, L)] * w)
                return 0

            lax.fori_loop(0, CHUNK // L, group_body, 0)
            pltpu.sync_copy(exbuf, den_sh.at[dst_idx], add=True)
            pltpu.sync_copy(vrows, agg_sh.at[dst_idx], add=True)

        return 0

    lax.fori_loop(0, NT, chunk_body, 0)
    plsc.subcore_barrier()

    pltpu.sync_copy(agg_sh.at[pl.ds(start, 640)],
                    agg_out.at[c].at[pl.ds(start, 640)])
    pltpu.sync_copy(den_sh.at[pl.ds(s * 640, 640)],
                    den_out.at[c].at[pl.ds(s * 640, 640)])


def _edge_pass(q, k, v, src, dst):
    mesh = plsc.VectorSubcoreMesh(core_axis_name="c", subcore_axis_name="s")
    call = pl.kernel(
        _edge_body,
        out_type=(jax.ShapeDtypeStruct((NC, N, D), FP32),
                  jax.ShapeDtypeStruct((NC, DEN_PAD), FP32)),
        mesh=mesh,
        compiler_params=pltpu.CompilerParams(needs_layout_passes=False),
        scratch_types=[
            pltpu.VMEM((CHUNK,), jnp.int32),
            pltpu.VMEM((CHUNK,), jnp.int32),
            pltpu.VMEM((CHUNK, D), FP32),
            pltpu.VMEM((CHUNK, D), FP32),
            pltpu.VMEM((CHUNK, D), FP32),
            pltpu.VMEM((CHUNK,), FP32),
            pltpu.VMEM((L * L,), FP32),
            pltpu.VMEM((640,), FP32),
            pltpu.VMEM_SHARED((N, D), FP32),
            pltpu.VMEM_SHARED((DEN_PAD,), FP32),
            pltpu.SemaphoreType.DMA,
            pltpu.SemaphoreType.DMA,
            pltpu.SemaphoreType.DMA,
        ],
    )
    return call(q, k, v, src, dst)


def _finish_body(aggp_ref, denp_ref, skip_ref, batch_ref, out_ref):
    agg = aggp_ref[0] + aggp_ref[1]
    den = denp_ref[0] + denp_ref[1] + 1e-16
    node = jax.nn.relu(agg / den + skip_ref[...])
    onehot = (batch_ref[...] ==
              lax.broadcasted_iota(jnp.int32, (G, N), 0)).astype(FP32)
    counts = jnp.sum(onehot, axis=1, keepdims=True)
    pooled = jnp.dot(onehot, node, preferred_element_type=FP32)
    out_ref[...] = pooled / jnp.maximum(counts, 1.0)


def _finish(agg_p, den_p, skip, batch):
    return pl.pallas_call(
        _finish_body,
        out_shape=jax.ShapeDtypeStruct((G, D), FP32),
    )(agg_p, den_p[:, :N].reshape(NC, N, 1), skip, batch.reshape(1, N))


def kernel(x, edge_index, batch, Wq, bq, Wk, bk, Wv, bv, Ws, bs):
    src = edge_index[0]
    dst = edge_index[1]
    q, k, v, skip = _project(x, Wq, Wk, Wv, Ws, bq, bk, bv, bs)
    agg_p, den_p = _edge_pass(q, k, v, src, dst)
    return _finish(agg_p, den_p, skip, batch)


# speedup vs baseline: 15.5918x; 1.3380x over previous
import functools

import jax
import jax.numpy as jnp
from jax import lax
from jax.experimental import pallas as pl
from jax.experimental.pallas import tpu as pltpu
from jax.experimental.pallas import tpu_sc as plsc

N = 10000
E = 320000
D = 128
G = 64
NC = 2
NS = 16
L = 16
NW = NC * NS
CHUNK = 64
NCHUNKS = E // CHUNK
NTW = -(-NCHUNKS // NW)
NFULL = NCHUNKS - (NTW - 1) * NW
NGRP = CHUNK // L
SPAN = 624
DEN_PAD = 640 * NS
SCALE = 1.0 / (D ** 0.5)
FP32 = jnp.float32


def _proj_body(x_ref, wq, wk, wv, ws, bq, bk, bv, bs, q_o, k_o, v_o, s_o):
    xb = x_ref[...]
    q_o[...] = jnp.dot(xb, wq[...], preferred_element_type=FP32) + bq[...]
    k_o[...] = jnp.dot(xb, wk[...], preferred_element_type=FP32) + bk[...]
    v_o[...] = jnp.dot(xb, wv[...], preferred_element_type=FP32) + bv[...]
    s_o[...] = jnp.dot(xb, ws[...], preferred_element_type=FP32) + bs[...]


def _project(x, Wq, Wk, Wv, Ws, bq, bk, bv, bs):
    BR = 1000
    grid = (N // BR,)
    row_spec = pl.BlockSpec((BR, D), lambda i: (i, 0))
    w_spec = pl.BlockSpec((D, D), lambda i: (0, 0))
    b_spec = pl.BlockSpec((1, D), lambda i: (0, 0))
    out = jax.ShapeDtypeStruct((N, D), FP32)
    return pl.pallas_call(
        _proj_body,
        grid=grid,
        in_specs=[row_spec, w_spec, w_spec, w_spec, w_spec,
                  b_spec, b_spec, b_spec, b_spec],
        out_specs=[row_spec, row_spec, row_spec, row_spec],
        out_shape=[out, out, out, out],
    )(x, Wq, Wk, Wv, Ws, bq.reshape(1, D), bk.reshape(1, D),
      bv.reshape(1, D), bs.reshape(1, D))


def _edge_body(q_hbm, k_hbm, v_hbm, src_hbm, dst_hbm,
               agg_out, den_out,
               src_idx, dst_idx, qrows, krows, vrows,
               exbuf, partials, zflat,
               agg_sh, den_sh, gsem, vsem, ssem):
    c = lax.axis_index("c")
    s = lax.axis_index("s")
    wid = s * NC + c

    zero16 = jnp.zeros((L,), FP32)

    def _zrow(r, _):
        for i in range(D // L):
            vrows[r, pl.ds(i * L, L)] = zero16
        return 0

    lax.fori_loop(0, CHUNK, _zrow, 0)
    for i in range(640 // L):
        zflat[pl.ds(i * L, L)] = zero16

    start = s * SPAN
    for j in range(640 // CHUNK):
        pltpu.sync_copy(vrows,
                        agg_sh.at[pl.ds(start + j * CHUNK, CHUNK)])
    pltpu.sync_copy(zflat, den_sh.at[pl.ds(s * 640, 640)])
    plsc.subcore_barrier()

    lane_iota = lax.iota(jnp.int32, L)

    def issue_gathers(b, cid):
        base = cid * CHUNK
        pltpu.sync_copy(src_hbm.at[pl.ds(base, CHUNK)], src_idx.at[b])
        pltpu.sync_copy(dst_hbm.at[pl.ds(base, CHUNK)], dst_idx.at[b])
        pltpu.async_copy(q_hbm.at[dst_idx.at[b]], qrows.at[b], gsem)
        pltpu.async_copy(k_hbm.at[src_idx.at[b]], krows.at[b], gsem)

    def drain_gathers(b):
        pltpu.make_async_copy(q_hbm.at[dst_idx.at[b]], qrows.at[b],
                              gsem).wait()
        pltpu.make_async_copy(k_hbm.at[src_idx.at[b]], krows.at[b],
                              gsem).wait()

    def issue_scatters(b):
        pltpu.async_copy(exbuf.at[b], den_sh.at[dst_idx.at[b]], ssem,
                         add=True)
        pltpu.async_copy(vrows, agg_sh.at[dst_idx.at[b]], ssem,
                         add=True)

    def drain_scatters(b):
        pltpu.make_async_copy(exbuf.at[b], den_sh.at[dst_idx.at[b]],
                              ssem).wait()
        pltpu.make_async_copy(vrows, agg_sh.at[dst_idx.at[b]],
                              ssem).wait()

    def compute(b):
        def group_body(g, _):
            gb = g * L
            for j in range(L):
                e = gb + j
                acc = qrows[b, e, pl.ds(0, L)] * krows[b, e, pl.ds(0, L)]
                for i in range(1, D // L):
                    acc = acc + (qrows[b, e, pl.ds(i * L, L)]
                                 * krows[b, e, pl.ds(i * L, L)])
                plsc.store_scatter(partials, [lane_iota * L + j], acc)
            score = partials[pl.ds(0, L)]
            for l in range(1, L):
                score = score + partials[pl.ds(l * L, L)]
            ex = jnp.exp(jnp.clip(score * SCALE, -60.0, 60.0))
            exbuf[b, pl.ds(gb, L)] = ex
            for j in range(L):
                e = gb + j
                w = plsc.load_gather(
                    exbuf.at[b], [jnp.full((L,), e, jnp.int32)])
                for i in range(D // L):
                    vrows[e, pl.ds(i * L, L)] = (
                        vrows[e, pl.ds(i * L, L)] * w)
            return 0

        lax.fori_loop(0, NGRP, group_body, 0)

    issue_gathers(0, wid)

    def chunk_body(t, _):
        cid = wid + t * NW
        b = lax.rem(t, 2)
        nb = 1 - b

        @pl.when(cid < NCHUNKS)
        def _():
            @pl.when(t >= 1)
            def _():
                drain_scatters(nb)

            pltpu.async_copy(v_hbm.at[src_idx.at[b]], vrows, vsem)
            drain_gathers(b)

            @pl.when(cid + NW < NCHUNKS)
            def _():
                issue_gathers(nb, cid + NW)

            pltpu.make_async_copy(v_hbm.at[src_idx.at[b]], vrows,
                                  vsem).wait()
            compute(b)
            issue_scatters(b)

        return 0

    lax.fori_loop(0, NTW, chunk_body, 0)

    @pl.when(wid < NFULL)
    def _():
        drain_scatters((NTW - 1) % 2)

    @pl.when(wid >= NFULL)
    def _():
        drain_scatters((NTW - 2) % 2)

    plsc.subcore_barrier()

    pltpu.sync_copy(agg_sh.at[pl.ds(start, 640)],
                    agg_out.at[c].at[pl.ds(start, 640)])
    pltpu.sync_copy(den_sh.at[pl.ds(s * 640, 640)],
                    den_out.at[c].at[pl.ds(s * 640, 640)])


def _edge_pass(q, k, v, src, dst):
    mesh = plsc.VectorSubcoreMesh(core_axis_name="c", subcore_axis_name="s")
    call = pl.kernel(
        _edge_body,
        out_type=(jax.ShapeDtypeStruct((NC, N, D), FP32),
                  jax.ShapeDtypeStruct((NC, DEN_PAD), FP32)),
        mesh=mesh,
        compiler_params=pltpu.CompilerParams(needs_layout_passes=False),
        scratch_types=[
            pltpu.VMEM((2, CHUNK), jnp.int32),
            pltpu.VMEM((2, CHUNK), jnp.int32),
            pltpu.VMEM((2, CHUNK, D), FP32),
            pltpu.VMEM((2, CHUNK, D), FP32),
            pltpu.VMEM((CHUNK, D), FP32),
            pltpu.VMEM((2, CHUNK), FP32),
            pltpu.VMEM((L * L,), FP32),
            pltpu.VMEM((640,), FP32),
            pltpu.VMEM_SHARED((N, D), FP32),
            pltpu.VMEM_SHARED((DEN_PAD,), FP32),
            pltpu.SemaphoreType.DMA,
            pltpu.SemaphoreType.DMA,
            pltpu.SemaphoreType.DMA,
        ],
    )
    return call(q, k, v, src, dst)


def _finish_body(aggp_ref, denp_ref, skip_ref, batch_ref, out_ref):
    agg = aggp_ref[0] + aggp_ref[1]
    den = denp_ref[0] + denp_ref[1] + 1e-16
    node = jax.nn.relu(agg / den + skip_ref[...])
    onehot = (batch_ref[...] ==
              lax.broadcasted_iota(jnp.int32, (G, N), 0)).astype(FP32)
    counts = jnp.sum(onehot, axis=1, keepdims=True)
    pooled = jnp.dot(onehot, node, preferred_element_type=FP32)
    out_ref[...] = pooled / jnp.maximum(counts, 1.0)


def _finish(agg_p, den_p, skip, batch):
    return pl.pallas_call(
        _finish_body,
        out_shape=jax.ShapeDtypeStruct((G, D), FP32),
    )(agg_p, den_p[:, :N].reshape(NC, N, 1), skip, batch.reshape(1, N))


def kernel(x, edge_index, batch, Wq, bq, Wk, bk, Wv, bv, Ws, bs):
    src = edge_index[0]
    dst = edge_index[1]
    q, k, v, skip = _project(x, Wq, Wk, Wv, Ws, bq, bk, bv, bs)
    agg_p, den_p = _edge_pass(q, k, v, src, dst)
    return _finish(agg_p, den_p, skip, batch)
